# Initial kernel scaffold; baseline (speedup 1.0000x reference)
import jax
import jax.numpy as jnp
from jax.experimental import pallas as pl


def kernel(x, table, pos_encoding):
    raise NotImplementedError("write your pallas kernel here")


# speedup vs baseline: 2.2656x; 2.2656x over previous
import functools
import math

import jax
import jax.numpy as jnp
from jax import lax
from jax.experimental import pallas as pl
from jax.experimental.pallas import tpu as pltpu
from jax.experimental.pallas import tpu_sc as plsc

_B = 4096
_L = 200
_D = 64
_LANES = 16
_NC = 2
_NS = 16
_NW = _NC * _NS
_ROWS_PER_W = _B // _NW
_GCHUNK = 100
_SCALE = math.sqrt(_D)


def _tec_body(x_hbm, table_hbm, pos_hbm, out_hbm, idx_v, pos_v, rows_v, sem):
    wid = lax.axis_index("s") * _NC + lax.axis_index("c")
    base = wid * _ROWS_PER_W

    pltpu.sync_copy(pos_hbm, pos_v)

    def row_step(r, carry):
        row = base + r
        pltpu.sync_copy(x_hbm.at[row], idx_v)
        cp0 = pltpu.async_copy(
            table_hbm.at[idx_v.at[0]], rows_v.at[pl.ds(0, _GCHUNK)], sem)
        cp1 = pltpu.async_copy(
            table_hbm.at[idx_v.at[1]], rows_v.at[pl.ds(_GCHUNK, _GCHUNK)], sem)
        cp0.wait()
        cp1.wait()

        def compute(l, c):
            for j in range(_D // _LANES):
                sl = pl.ds(j * _LANES, _LANES)
                rows_v[l, sl] = rows_v[l, sl] * _SCALE + pos_v[l, sl]
            return c

        lax.fori_loop(0, _L, compute, 0, unroll=2)
        pltpu.sync_copy(rows_v, out_hbm.at[row])
        return carry

    lax.fori_loop(0, _ROWS_PER_W, row_step, 0)


@jax.jit
def _run(x2, table, pos_encoding):
    mesh = plsc.VectorSubcoreMesh(core_axis_name="c", subcore_axis_name="s")
    f = pl.kernel(
        _tec_body,
        out_type=jax.ShapeDtypeStruct((_B, _L, _D), jnp.float32),
        mesh=mesh,
        scratch_types=[
            pltpu.VMEM((2, _GCHUNK), jnp.int32),
            pltpu.VMEM((_L, _D), jnp.float32),
            pltpu.VMEM((_L, _D), jnp.float32),
            pltpu.SemaphoreType.DMA,
        ],
        compiler_params=pltpu.CompilerParams(use_tc_tiling_on_sc=False),
    )
    return f(x2, table, pos_encoding)


def kernel(x, table, pos_encoding):
    x2 = x.astype(jnp.int32).reshape(_B, 2, _GCHUNK)
    return _run(x2, table, pos_encoding)


# speedup vs baseline: 3.4298x; 1.5139x over previous
import functools
import math

import jax
import jax.numpy as jnp
from jax import lax
from jax.experimental import pallas as pl
from jax.experimental.pallas import tpu as pltpu
from jax.experimental.pallas import tpu_sc as plsc

_B = 4096
_L = 200
_D = 64
_LANES = 16
_NC = 2
_NS = 16
_NW = _NC * _NS
_ROWS_PER_W = _B // _NW
_G = 100
_NB = 4
_PF = 2
_SCALE = math.sqrt(_D)


def _tec_body(x_hbm, table_hbm, pos_hbm, out_hbm, idx_v, pos_v, rows_v,
              gsem, osem):
    wid = lax.axis_index("s") * _NC + lax.axis_index("c")
    base = wid * _ROWS_PER_W

    pltpu.sync_copy(pos_hbm, pos_v)
    pltpu.sync_copy(x_hbm.at[pl.ds(base, _ROWS_PER_W)], idx_v)

    def gather_row(r, b):
        pltpu.async_copy(table_hbm.at[idx_v.at[r, 0]],
                         rows_v.at[b, pl.ds(0, _G)], gsem.at[b])
        pltpu.async_copy(table_hbm.at[idx_v.at[r, 1]],
                         rows_v.at[b, pl.ds(_G, _G)], gsem.at[b])

    def wait_gathers(b):
        for h in range(2):
            pltpu.make_async_copy(table_hbm.at[idx_v.at[0, h]],
                                  rows_v.at[b, pl.ds(h * _G, _G)],
                                  gsem.at[b]).wait()

    def wait_store(b):
        pltpu.make_async_copy(rows_v.at[b], out_hbm.at[base],
                              osem.at[b]).wait()

    for r in range(_PF):
        gather_row(r, r)

    @pl.loop(0, _ROWS_PER_W, step=_NB)
    def row_block(k):
        for b in range(_NB):
            row = k + b
            j = row + _PF
            bj = (b + _PF) % _NB

            @pl.when(j < _ROWS_PER_W)
            def _prefetch():
                @pl.when(j >= _NB)
                def _drain():
                    wait_store(bj)
                gather_row(j, bj)

            wait_gathers(b)

            def compute(l, c):
                for v in range(_D // _LANES):
                    sl = pl.ds(v * _LANES, _LANES)
                    rows_v[b, l, sl] = rows_v[b, l, sl] * _SCALE + pos_v[l, sl]
                return c

            lax.fori_loop(0, _L, compute, 0, unroll=2)
            pltpu.async_copy(rows_v.at[b], out_hbm.at[base + row], osem.at[b])

    for b in range(_NB):
        wait_store(b)


@jax.jit
def _run(x2, table, pos_encoding):
    mesh = plsc.VectorSubcoreMesh(core_axis_name="c", subcore_axis_name="s")
    f = pl.kernel(
        _tec_body,
        out_type=jax.ShapeDtypeStruct((_B, _L, _D), jnp.float32),
        mesh=mesh,
        scratch_types=[
            pltpu.VMEM((_ROWS_PER_W, 2, _G), jnp.int32),
            pltpu.VMEM((_L, _D), jnp.float32),
            pltpu.VMEM((_NB, _L, _D), jnp.float32),
            pltpu.SemaphoreType.DMA((_NB,)),
            pltpu.SemaphoreType.DMA((_NB,)),
        ],
        compiler_params=pltpu.CompilerParams(use_tc_tiling_on_sc=False),
    )
    return f(x2, table, pos_encoding)


def kernel(x, table, pos_encoding):
    x2 = x.astype(jnp.int32).reshape(_B, 2, _G)
    return _run(x2, table, pos_encoding)


# speedup vs baseline: 3.4843x; 1.0159x over previous
import functools
import math

import jax
import jax.numpy as jnp
from jax import lax
from jax.experimental import pallas as pl
from jax.experimental.pallas import tpu as pltpu
from jax.experimental.pallas import tpu_sc as plsc

_B = 4096
_L = 200
_D = 64
_LANES = 16
_NC = 2
_NS = 16
_NW = _NC * _NS
_ROWS_PER_W = _B // _NW
_SPLITS = ((0, 104), (104, 96))
_NB = 4
_PF = 2
_SCALE = math.sqrt(_D)


def _tec_body(x_hbm, table_hbm, pos_hbm, out_hbm, idx_v, pos_v, rows_v,
              gsem, osem):
    wid = lax.axis_index("s") * _NC + lax.axis_index("c")
    base = wid * _ROWS_PER_W

    pltpu.sync_copy(pos_hbm, pos_v)
    pltpu.sync_copy(x_hbm.at[pl.ds(base, _ROWS_PER_W)], idx_v)

    def gather_row(r, b):
        for off, n in _SPLITS:
            pltpu.async_copy(table_hbm.at[idx_v.at[r, pl.ds(off, n)]],
                             rows_v.at[b, pl.ds(off, n)], gsem.at[b])

    def wait_gathers(b):
        for off, n in _SPLITS:
            pltpu.make_async_copy(table_hbm.at[idx_v.at[0, pl.ds(off, n)]],
                                  rows_v.at[b, pl.ds(off, n)],
                                  gsem.at[b]).wait()

    def wait_store(b):
        pltpu.make_async_copy(rows_v.at[b], out_hbm.at[base],
                              osem.at[b]).wait()

    for r in range(_PF):
        gather_row(r, r)

    @pl.loop(0, _ROWS_PER_W, step=_NB)
    def row_block(k):
        for b in range(_NB):
            row = k + b
            j = row + _PF
            bj = (b + _PF) % _NB

            @pl.when(j < _ROWS_PER_W)
            def _prefetch():
                @pl.when(j >= _NB)
                def _drain():
                    wait_store(bj)
                gather_row(j, bj)

            wait_gathers(b)

            def compute(l, c):
                for v in range(_D // _LANES):
                    sl = pl.ds(v * _LANES, _LANES)
                    rows_v[b, l, sl] = rows_v[b, l, sl] * _SCALE + pos_v[l, sl]
                return c

            lax.fori_loop(0, _L, compute, 0, unroll=2)
            pltpu.async_copy(rows_v.at[b], out_hbm.at[base + row], osem.at[b])

    for b in range(_NB):
        wait_store(b)


@jax.jit
def _run(x, table, pos_encoding):
    mesh = plsc.VectorSubcoreMesh(core_axis_name="c", subcore_axis_name="s")
    f = pl.kernel(
        _tec_body,
        out_type=jax.ShapeDtypeStruct((_B, _L, _D), jnp.float32),
        mesh=mesh,
        scratch_types=[
            pltpu.VMEM((_ROWS_PER_W, _L), jnp.int32),
            pltpu.VMEM((_L, _D), jnp.float32),
            pltpu.VMEM((_NB, _L, _D), jnp.float32),
            pltpu.SemaphoreType.DMA((_NB,)),
            pltpu.SemaphoreType.DMA((_NB,)),
        ],
        compiler_params=pltpu.CompilerParams(use_tc_tiling_on_sc=False),
    )
    return f(x, table, pos_encoding)


def kernel(x, table, pos_encoding):
    return _run(x.astype(jnp.int32), table, pos_encoding)
